# baseline (speedup 1.0000x reference)
import jax
import jax.numpy as jnp
from jax.experimental import pallas as pl
from jax.experimental.pallas import tpu as pltpu

_N_CLASSES = 1000
_B_BLOCK = 32
_NBUF = 8


def _perm(i):
    return (i % 8) * 16 + i // 8


def _onehot_body(x_ref, o_ref, vmem, sem):
    i = pl.program_id(0)
    nb = pl.num_programs(0)
    slot = jax.lax.rem(i, _NBUF)

    def copy(j, s):
        return pltpu.make_async_copy(
            vmem.at[s], o_ref.at[pl.ds(_perm(j) * _B_BLOCK, _B_BLOCK)], sem.at[s]
        )

    @pl.when(i >= _NBUF)
    def _wait_prev():
        copy(i - _NBUF, slot).wait()

    idx = x_ref[...]
    iota = jax.lax.broadcasted_iota(
        jnp.int32, (_B_BLOCK, idx.shape[1], _N_CLASSES), 2
    )
    vmem[slot] = (idx[:, :, None] == iota).astype(jnp.float32)
    for k in range(_NBUF):
        @pl.when(slot == k)
        def _start():
            copy(i, k).start(priority=k % 2)

    @pl.when(i == nb - 1)
    def _drain():
        for k in range(_NBUF):
            j = i - (_NBUF - 1) + k
            copy(j, jax.lax.rem(j, _NBUF)).wait()


def kernel(x):
    b, s = x.shape
    nb = b // _B_BLOCK
    return pl.pallas_call(
        _onehot_body,
        grid=(nb,),
        in_specs=[pl.BlockSpec((_B_BLOCK, s), lambda i: (_perm(i), 0))],
        out_specs=pl.BlockSpec(memory_space=pl.MemorySpace.ANY),
        out_shape=jax.ShapeDtypeStruct((b, s, _N_CLASSES), jnp.float32),
        scratch_shapes=[
            pltpu.VMEM((_NBUF, _B_BLOCK, s, _N_CLASSES), jnp.float32),
            pltpu.SemaphoreType.DMA((_NBUF,)),
        ],
    )(x)
